# baseline (speedup 1.0000x reference)
import functools

import jax
import jax.numpy as jnp
from jax import lax
from jax.experimental import pallas as pl
from jax.experimental.pallas import tpu as pltpu
from jax.experimental.pallas import tpu_sc as plsc

VOCAB = 1000000
EMBED_DIM = 64
BATCH = 4096
P_LEN = 50

NUM_WORKERS = 32
B_PER_WORKER = BATCH // NUM_WORKERS

TBLK = 16384
QBLK = TBLK // 4
TGRID = (VOCAB + TBLK - 1) // TBLK
TROWS = TGRID * 2 * QBLK

NCHUNK = B_PER_WORKER
NBUF = 4
LOOKAHEAD = 2


def _tc_transpose(wv_t):
    def body(i_ref, o_ref):
        x = i_ref[...]
        x4 = jnp.concatenate(
            [x[:, q * QBLK:(q + 1) * QBLK] for q in range(4)], axis=0
        )
        eye = jnp.eye(256, dtype=jnp.float32)
        t = jax.lax.dot_general(
            x4, eye, (((0,), (0,)), ((), ())),
            precision=jax.lax.Precision.HIGHEST,
        )
        o_ref[0:QBLK, :] = t[:, 0:128]
        o_ref[QBLK:2 * QBLK, :] = t[:, 128:256]

    return pl.pallas_call(
        body,
        grid=(TGRID,),
        in_specs=[pl.BlockSpec((EMBED_DIM, TBLK), lambda g: (0, g))],
        out_specs=pl.BlockSpec((2 * QBLK, 128), lambda g: (g, 0)),
        out_shape=jax.ShapeDtypeStruct((TROWS, 128), jnp.float32),
    )(wv_t)


def _build_gather():
    mesh = plsc.VectorSubcoreMesh(core_axis_name="c", subcore_axis_name="s")

    scratch = [pltpu.VMEM((B_PER_WORKER, P_LEN), jnp.int32)]
    scratch += [pltpu.VMEM((P_LEN, EMBED_DIM), jnp.float32) for _ in range(NBUF)]
    scratch += [pltpu.SemaphoreType.DMA for _ in range(2 * NBUF)]

    @functools.partial(
        pl.kernel,
        mesh=mesh,
        compiler_params=pltpu.CompilerParams(use_tc_tiling_on_sc=False),
        out_type=jax.ShapeDtypeStruct((BATCH, P_LEN, EMBED_DIM), jnp.float32),
        scratch_types=scratch,
    )
    def gather_kernel(table_hbm, idx_hbm, out_hbm, idx_v, *bufs_and_sems):
        rows = bufs_and_sems[:NBUF]
        sem_g = bufs_and_sems[NBUF:2 * NBUF]
        sem_w = bufs_and_sems[2 * NBUF:]

        wid = lax.axis_index("s") * 2 + lax.axis_index("c")
        base = wid * B_PER_WORKER
        pltpu.sync_copy(idx_hbm.at[pl.ds(base, B_PER_WORKER)], idx_v)

        def fire_gather(f, bf):
            pltpu.async_copy(table_hbm.at[idx_v.at[f]], rows[bf], sem_g[bf])

        def wait_gather(bf):
            pltpu.make_async_copy(
                table_hbm.at[idx_v.at[0]], rows[bf], sem_g[bf]
            ).wait()

        def fire_write(g, b):
            pltpu.async_copy(rows[b], out_hbm.at[base + g], sem_w[b])

        def wait_write(b):
            pltpu.make_async_copy(rows[b], out_hbm.at[base], sem_w[b]).wait()

        for b in range(LOOKAHEAD):
            fire_gather(b, b)

        def body(t, carry):
            for b in range(NBUF):
                g = t * NBUF + b
                f = g + LOOKAHEAD
                bf = (b + LOOKAHEAD) % NBUF

                @pl.when(f < NCHUNK)
                def _():
                    @pl.when(f >= NBUF)
                    def _():
                        wait_write(bf)

                    fire_gather(f, bf)

                wait_gather(b)
                fire_write(g, b)
            return carry

        lax.fori_loop(0, NCHUNK // NBUF, body, 0)

        for b in range(NBUF):
            wait_write(b)

    return gather_kernel


_gather = _build_gather()


def kernel(x, pw_idxs, qw_idxs, p_mask, q_mask, word_vectors):
    tbl2 = _tc_transpose(word_vectors.T)
    view = tbl2.reshape(2 * TROWS, EMBED_DIM)
    idx = x.astype(jnp.int32)
    l = idx % TBLK
    q = l // QBLK
    row = 2 * QBLK * (idx // TBLK) + QBLK * (q // 2) + (l % QBLK)
    vrow = 2 * row + (q % 2)
    return _gather(view, vrow)


# speedup vs baseline: 1.0240x; 1.0240x over previous
import functools

import jax
import jax.numpy as jnp
from jax import lax
from jax.experimental import pallas as pl
from jax.experimental.pallas import tpu as pltpu
from jax.experimental.pallas import tpu_sc as plsc

VOCAB = 1000000
EMBED_DIM = 64
BATCH = 4096
P_LEN = 50

NUM_WORKERS = 32
B_PER_WORKER = BATCH // NUM_WORKERS

TBLK = 16384
QBLK = TBLK // 4
TGRID = (VOCAB + TBLK - 1) // TBLK
TROWS = TGRID * 2 * QBLK

NCHUNK = B_PER_WORKER
NBUF = 8
LOOKAHEAD = 4


def _tc_transpose(wv_t):
    def body(i_ref, o_ref):
        x = i_ref[...]
        x4 = jnp.concatenate(
            [x[:, q * QBLK:(q + 1) * QBLK] for q in range(4)], axis=0
        )
        eye = jnp.eye(256, dtype=jnp.float32)
        t = jax.lax.dot_general(
            x4, eye, (((0,), (0,)), ((), ())),
            precision=jax.lax.Precision.HIGHEST,
        )
        o_ref[0:QBLK, :] = t[:, 0:128]
        o_ref[QBLK:2 * QBLK, :] = t[:, 128:256]

    return pl.pallas_call(
        body,
        grid=(TGRID,),
        in_specs=[pl.BlockSpec((EMBED_DIM, TBLK), lambda g: (0, g))],
        out_specs=pl.BlockSpec((2 * QBLK, 128), lambda g: (g, 0)),
        out_shape=jax.ShapeDtypeStruct((TROWS, 128), jnp.float32),
    )(wv_t)


def _build_gather():
    mesh = plsc.VectorSubcoreMesh(core_axis_name="c", subcore_axis_name="s")

    scratch = [pltpu.VMEM((B_PER_WORKER, P_LEN), jnp.int32)]
    scratch += [pltpu.VMEM((P_LEN, EMBED_DIM), jnp.float32) for _ in range(NBUF)]
    scratch += [pltpu.SemaphoreType.DMA for _ in range(2 * NBUF)]

    @functools.partial(
        pl.kernel,
        mesh=mesh,
        compiler_params=pltpu.CompilerParams(use_tc_tiling_on_sc=False),
        out_type=jax.ShapeDtypeStruct((BATCH, P_LEN, EMBED_DIM), jnp.float32),
        scratch_types=scratch,
    )
    def gather_kernel(table_hbm, idx_hbm, out_hbm, idx_v, *bufs_and_sems):
        rows = bufs_and_sems[:NBUF]
        sem_g = bufs_and_sems[NBUF:2 * NBUF]
        sem_w = bufs_and_sems[2 * NBUF:]

        wid = lax.axis_index("s") * 2 + lax.axis_index("c")
        base = wid * B_PER_WORKER
        pltpu.sync_copy(idx_hbm.at[pl.ds(base, B_PER_WORKER)], idx_v)

        def fire_gather(f, bf):
            pltpu.async_copy(table_hbm.at[idx_v.at[f]], rows[bf], sem_g[bf])

        def wait_gather(bf):
            pltpu.make_async_copy(
                table_hbm.at[idx_v.at[0]], rows[bf], sem_g[bf]
            ).wait()

        def fire_write(g, b):
            pltpu.async_copy(rows[b], out_hbm.at[base + g], sem_w[b])

        def wait_write(b):
            pltpu.make_async_copy(rows[b], out_hbm.at[base], sem_w[b]).wait()

        for b in range(LOOKAHEAD):
            fire_gather(b, b)

        def body(t, carry):
            for b in range(NBUF):
                g = t * NBUF + b
                f = g + LOOKAHEAD
                bf = (b + LOOKAHEAD) % NBUF

                @pl.when(f < NCHUNK)
                def _():
                    @pl.when(f >= NBUF)
                    def _():
                        wait_write(bf)

                    fire_gather(f, bf)

                wait_gather(b)
                fire_write(g, b)
            return carry

        lax.fori_loop(0, NCHUNK // NBUF, body, 0)

        for b in range(NBUF):
            wait_write(b)

    return gather_kernel


_gather = _build_gather()


def kernel(x, pw_idxs, qw_idxs, p_mask, q_mask, word_vectors):
    tbl2 = _tc_transpose(word_vectors.T)
    view = tbl2.reshape(2 * TROWS, EMBED_DIM)
    idx = x.astype(jnp.int32)
    l = idx % TBLK
    q = l // QBLK
    row = 2 * QBLK * (idx // TBLK) + QBLK * (q // 2) + (l % QBLK)
    vrow = 2 * row + (q % 2)
    return _gather(view, vrow)
